# baseline (speedup 1.0000x reference)
import functools

import jax
import jax.numpy as jnp
from jax import lax
from jax.experimental import pallas as pl
from jax.experimental.pallas import tpu as pltpu
from jax.experimental.pallas import tpu_sc as plsc

N = 10000
E = 320000
NC = 2
NS = 16
NW = NC * NS
EPW = E // NW
NCHUNK = EPW // 16
NROW = (EPW + 127) // 128
RB = 400
RGRID = N // RB

@functools.cache
def _mesh():
    return plsc.VectorSubcoreMesh(
        core_axis_name="c", subcore_axis_name="s", num_cores=NC, num_subcores=NS
    )


def _wid():
    return lax.axis_index("s") * NC + lax.axis_index("c")


def _edge_chunk(src_v, dst_v, j, c):
    e = j * 8 + c
    off = jnp.minimum(e, NCHUNK - 1) * 16
    s = src_v[pl.ds(off, 16)]
    d = dst_v[pl.ds(off, 16)]
    return s, d, off


def _body_a(src_h, dst_h, px_h, py_h, pz_h, z_h,
            src_v, dst_v, px_v, py_v, pz_v, idx2, val2, sem):
    base = _wid() * EPW
    pltpu.sync_copy(src_h.at[pl.ds(base, EPW)], src_v)
    pltpu.sync_copy(dst_h.at[pl.ds(base, EPW)], dst_v)
    pltpu.sync_copy(px_h, px_v)
    pltpu.sync_copy(py_h, py_v)
    pltpu.sync_copy(pz_h, pz_v)

    @pl.loop(0, NROW)
    def _compute(j):
        for c in range(8):
            s, d, _ = _edge_chunk(src_v, dst_v, j, c)
            dx = plsc.load_gather(px_v, [d]) - plsc.load_gather(px_v, [s])
            dy = plsc.load_gather(py_v, [d]) - plsc.load_gather(py_v, [s])
            dz = plsc.load_gather(pz_v, [d]) - plsc.load_gather(pz_v, [s])
            w = 1.0 / (1.0 + dx * dx + dy * dy + dz * dz)
            idx2[j, pl.ds(c * 16, 16)] = s * N + d
            val2[j, pl.ds(c * 16, 16)] = w

    @pl.loop(0, NROW)
    def _fire(j):
        pltpu.async_copy(val2.at[j], z_h.at[idx2.at[j]], sem)

    @pl.loop(0, NROW)
    def _drain(j):
        pltpu.make_async_copy(val2.at[j], z_h.at[idx2.at[j]], sem).wait()


@functools.cache
def _kernel_a():
    return pl.kernel(
        _body_a,
        out_type=(),
        mesh=_mesh(),
        compiler_params=pltpu.CompilerParams(needs_layout_passes=False),
        scratch_types=[
            pltpu.VMEM((EPW,), jnp.int32),
            pltpu.VMEM((EPW,), jnp.int32),
            pltpu.VMEM((N,), jnp.float32),
            pltpu.VMEM((N,), jnp.float32),
            pltpu.VMEM((N,), jnp.float32),
            pltpu.VMEM((NROW, 128), jnp.int32),
            pltpu.VMEM((NROW, 128), jnp.float32),
            pltpu.SemaphoreType.DMA,
        ],
    )


RPT = 320


def _body_rs_sc(z_h, w_h, buf, wv, sem):
    lo = jnp.minimum(_wid() * RPT, N - RPT)
    lanes = lax.iota(jnp.int32, 16)

    @pl.loop(0, RPT // 16)
    def _grp(g):
        def _row(k, res):
            r = lo + g * 16 + k
            pltpu.sync_copy(z_h.at[pl.ds(r * N, N)], buf)

            def _acc(c, a):
                return a + buf[pl.ds(c * 16, 16)]

            acc = lax.fori_loop(0, N // 16, _acc,
                                jnp.zeros((16,), jnp.float32), unroll=8)
            return jnp.where(lanes == k, jnp.sum(acc), res)

        wv[pl.ds(g * 16, 16)] = lax.fori_loop(
            0, 16, _row, jnp.zeros((16,), jnp.float32))

    pltpu.sync_copy(wv, w_h.at[pl.ds(lo, RPT)])


@functools.cache
def _rowsum_sc():
    return pl.kernel(
        _body_rs_sc,
        out_type=jax.ShapeDtypeStruct((N,), jnp.float32),
        mesh=_mesh(),
        compiler_params=pltpu.CompilerParams(needs_layout_passes=False),
        scratch_types=[
            pltpu.VMEM((N,), jnp.float32),
            pltpu.VMEM((RPT,), jnp.float32),
            pltpu.SemaphoreType.DMA,
        ],
    )


FR = 128


def _body_fin(zf, zt, wc, wr, o):
    wcs = jnp.where(wc[...] > 0.0, wc[...], 1.0)
    wrs = jnp.where(wr[...] > 0.0, wr[...], 1.0)
    o[...] = 0.5 * (zf[...] / wcs + jnp.swapaxes(zt[...], 0, 1) / wrs)


_final = pl.pallas_call(
    _body_fin,
    out_shape=jax.ShapeDtypeStruct((N, N), jnp.float32),
    grid=(pl.cdiv(N, FR),),
    in_specs=[
        pl.BlockSpec((FR, N), lambda i: (i, 0)),
        pl.BlockSpec((N, FR), lambda i: (0, i)),
        pl.BlockSpec((FR, 1), lambda i: (i, 0)),
        pl.BlockSpec((1, N), lambda i: (0, 0)),
    ],
    out_specs=pl.BlockSpec((FR, N), lambda i: (i, 0)),
)


def kernel(features, pos, edges, faces):
    del features, faces
    src = edges[0].astype(jnp.int32)
    dst = edges[1].astype(jnp.int32)
    px = jnp.asarray(pos[:, 0], jnp.float32)
    py = jnp.asarray(pos[:, 1], jnp.float32)
    pz = jnp.asarray(pos[:, 2], jnp.float32)

    z_ref = jax.new_ref(jnp.zeros((N * N,), jnp.float32))
    _kernel_a()(src, dst, px, py, pz, z_ref)
    w = _rowsum_sc()(z_ref)
    z2 = jax.freeze(z_ref).reshape(N, N)
    return _final(z2, z2, w.reshape(N, 1), w.reshape(1, N))


# speedup vs baseline: 1.3327x; 1.3327x over previous
import functools

import jax
import jax.numpy as jnp
from jax import lax
from jax.experimental import pallas as pl
from jax.experimental.pallas import tpu as pltpu
from jax.experimental.pallas import tpu_sc as plsc

N = 10000
E = 320000
NC = 2
NS = 16
NW = NC * NS
EPW = E // NW
NCHUNK = EPW // 16
NROW = (EPW + 127) // 128
CWA = 128
ROWSA = (EPW + CWA - 1) // CWA
SUBA = CWA // 16
RB = 400
RGRID = N // RB

@functools.cache
def _mesh():
    return plsc.VectorSubcoreMesh(
        core_axis_name="c", subcore_axis_name="s", num_cores=NC, num_subcores=NS
    )


def _wid():
    return lax.axis_index("s") * NC + lax.axis_index("c")


def _edge_chunk(src_v, dst_v, j, c):
    e = j * 8 + c
    off = jnp.minimum(e, NCHUNK - 1) * 16
    s = src_v[pl.ds(off, 16)]
    d = dst_v[pl.ds(off, 16)]
    return s, d, off


def _body_a(src_h, dst_h, px_h, py_h, pz_h, z_h,
            src_v, dst_v, px_v, py_v, pz_v, idx2, val2, sem):
    base = _wid() * EPW
    pltpu.sync_copy(src_h.at[pl.ds(base, EPW)], src_v)
    pltpu.sync_copy(dst_h.at[pl.ds(base, EPW)], dst_v)
    pltpu.sync_copy(px_h, px_v)
    pltpu.sync_copy(py_h, py_v)
    pltpu.sync_copy(pz_h, pz_v)

    @pl.loop(0, ROWSA)
    def _compute(j):
        for c in range(SUBA):
            e = j * SUBA + c
            off = jnp.minimum(e, NCHUNK - 1) * 16
            s = src_v[pl.ds(off, 16)]
            d = dst_v[pl.ds(off, 16)]
            dx = plsc.load_gather(px_v, [d]) - plsc.load_gather(px_v, [s])
            dy = plsc.load_gather(py_v, [d]) - plsc.load_gather(py_v, [s])
            dz = plsc.load_gather(pz_v, [d]) - plsc.load_gather(pz_v, [s])
            w = 1.0 / (1.0 + dx * dx + dy * dy + dz * dz)
            idx2[j, pl.ds(c * 16, 16)] = s * N + d
            val2[j, pl.ds(c * 16, 16)] = w

    @pl.loop(0, ROWSA)
    def _fire(j):
        pltpu.async_copy(val2.at[j], z_h.at[idx2.at[j]], sem)

    @pl.loop(0, ROWSA)
    def _drain(j):
        pltpu.make_async_copy(val2.at[j], z_h.at[idx2.at[j]], sem).wait()


@functools.cache
def _kernel_a():
    return pl.kernel(
        _body_a,
        out_type=(),
        mesh=_mesh(),
        compiler_params=pltpu.CompilerParams(needs_layout_passes=False),
        scratch_types=[
            pltpu.VMEM((EPW,), jnp.int32),
            pltpu.VMEM((EPW,), jnp.int32),
            pltpu.VMEM((N,), jnp.float32),
            pltpu.VMEM((N,), jnp.float32),
            pltpu.VMEM((N,), jnp.float32),
            pltpu.VMEM((ROWSA, CWA), jnp.int32),
            pltpu.VMEM((ROWSA, CWA), jnp.float32),
            pltpu.SemaphoreType.DMA,
        ],
    )


def _body_rs(z_blk, o_blk):
    o_blk[...] = jnp.sum(z_blk[...], axis=1, keepdims=True)


_rowsum = pl.pallas_call(
    _body_rs,
    out_shape=jax.ShapeDtypeStruct((N, 1), jnp.float32),
    grid=(RGRID,),
    in_specs=[pl.BlockSpec((RB, N), lambda i: (i, 0))],
    out_specs=pl.BlockSpec((RB, 1), lambda i: (i, 0)),
)


FR = 256


def _body_fin(zf, zt, wc, wr, o):
    wcs = jnp.where(wc[...] > 0.0, wc[...], 1.0)
    wrs = jnp.where(wr[...] > 0.0, wr[...], 1.0)
    o[...] = 0.5 * (zf[...] / wcs + jnp.swapaxes(zt[...], 0, 1) / wrs)


_final = pl.pallas_call(
    _body_fin,
    out_shape=jax.ShapeDtypeStruct((N, N), jnp.float32),
    grid=(pl.cdiv(N, FR),),
    compiler_params=pltpu.CompilerParams(vmem_limit_bytes=127 * 1024 * 1024),
    in_specs=[
        pl.BlockSpec((FR, N), lambda i: (i, 0)),
        pl.BlockSpec((N, FR), lambda i: (0, i)),
        pl.BlockSpec((FR, 1), lambda i: (i, 0)),
        pl.BlockSpec((1, N), lambda i: (0, 0)),
    ],
    out_specs=pl.BlockSpec((FR, N), lambda i: (i, 0)),
)


def kernel(features, pos, edges, faces):
    del features, faces
    src = edges[0].astype(jnp.int32)
    dst = edges[1].astype(jnp.int32)
    px = jnp.asarray(pos[:, 0], jnp.float32)
    py = jnp.asarray(pos[:, 1], jnp.float32)
    pz = jnp.asarray(pos[:, 2], jnp.float32)

    z_ref = jax.new_ref(jnp.zeros((N * N,), jnp.float32))
    _kernel_a()(src, dst, px, py, pz, z_ref)
    z2 = jax.freeze(z_ref).reshape(N, N)
    w2 = _rowsum(z2)
    return _final(z2, z2, w2, w2.reshape(1, N))


# speedup vs baseline: 1.3867x; 1.0405x over previous
import functools

import jax
import jax.numpy as jnp
from jax import lax
from jax.experimental import pallas as pl
from jax.experimental.pallas import tpu as pltpu
from jax.experimental.pallas import tpu_sc as plsc

N = 10000
E = 320000
NC = 2
NS = 16
NW = NC * NS
EPW = E // NW
NCHUNK = EPW // 16
NROW = (EPW + 127) // 128
CWA = 128
ROWSA = (EPW + CWA - 1) // CWA
SUBA = CWA // 16
RB = 400
RGRID = N // RB

@functools.cache
def _mesh():
    return plsc.VectorSubcoreMesh(
        core_axis_name="c", subcore_axis_name="s", num_cores=NC, num_subcores=NS
    )


def _wid():
    return lax.axis_index("s") * NC + lax.axis_index("c")


def _edge_chunk(src_v, dst_v, j, c):
    e = j * 8 + c
    off = jnp.minimum(e, NCHUNK - 1) * 16
    s = src_v[pl.ds(off, 16)]
    d = dst_v[pl.ds(off, 16)]
    return s, d, off


def _body_a(src_h, dst_h, px_h, py_h, pz_h, z_h,
            src_v, dst_v, px_v, py_v, pz_v, idx2, val2, sem):
    base = _wid() * EPW
    pltpu.sync_copy(src_h.at[pl.ds(base, EPW)], src_v)
    pltpu.sync_copy(dst_h.at[pl.ds(base, EPW)], dst_v)
    pltpu.sync_copy(px_h, px_v)
    pltpu.sync_copy(py_h, py_v)
    pltpu.sync_copy(pz_h, pz_v)

    @pl.loop(0, ROWSA)
    def _compute(j):
        for c in range(SUBA):
            e = j * SUBA + c
            off = jnp.minimum(e, NCHUNK - 1) * 16
            s = src_v[pl.ds(off, 16)]
            d = dst_v[pl.ds(off, 16)]
            dx = plsc.load_gather(px_v, [d]) - plsc.load_gather(px_v, [s])
            dy = plsc.load_gather(py_v, [d]) - plsc.load_gather(py_v, [s])
            dz = plsc.load_gather(pz_v, [d]) - plsc.load_gather(pz_v, [s])
            w = 1.0 / (1.0 + dx * dx + dy * dy + dz * dz)
            idx2[j, pl.ds(c * 16, 16)] = s * N + d
            val2[j, pl.ds(c * 16, 16)] = w

    @pl.loop(0, ROWSA)
    def _fire(j):
        pltpu.async_copy(val2.at[j], z_h.at[idx2.at[j]], sem)

    @pl.loop(0, ROWSA)
    def _drain(j):
        pltpu.make_async_copy(val2.at[j], z_h.at[idx2.at[j]], sem).wait()


@functools.cache
def _kernel_a():
    return pl.kernel(
        _body_a,
        out_type=(),
        mesh=_mesh(),
        compiler_params=pltpu.CompilerParams(needs_layout_passes=False),
        scratch_types=[
            pltpu.VMEM((EPW,), jnp.int32),
            pltpu.VMEM((EPW,), jnp.int32),
            pltpu.VMEM((N,), jnp.float32),
            pltpu.VMEM((N,), jnp.float32),
            pltpu.VMEM((N,), jnp.float32),
            pltpu.VMEM((ROWSA, CWA), jnp.int32),
            pltpu.VMEM((ROWSA, CWA), jnp.float32),
            pltpu.SemaphoreType.DMA,
        ],
    )


def _body_rs(z_blk, o_blk):
    o_blk[...] = jnp.sum(z_blk[...], axis=1, keepdims=True)


_rowsum = pl.pallas_call(
    _body_rs,
    out_shape=jax.ShapeDtypeStruct((N, 1), jnp.float32),
    grid=(RGRID,),
    in_specs=[pl.BlockSpec((RB, N), lambda i: (i, 0))],
    out_specs=pl.BlockSpec((RB, 1), lambda i: (i, 0)),
)


RPT = 320


def _body_rs_sc(z_h, w_h, buf0, buf1, wv, sem0, sem1):
    lo = jnp.minimum(_wid() * RPT, N - RPT)
    lanes = lax.iota(jnp.int32, 16)
    bufs = (buf0, buf1)
    sems = (sem0, sem1)

    pltpu.async_copy(z_h.at[pl.ds(lo * N, N)], buf0, sem0)

    @pl.loop(0, RPT // 16)
    def _grp(g):
        res = jnp.zeros((16,), jnp.float32)
        for k in range(16):
            i = g * 16 + k
            nxt = i + 1

            @pl.when(nxt < RPT)
            def _():
                pltpu.async_copy(z_h.at[pl.ds((lo + nxt) * N, N)],
                                 bufs[(k + 1) % 2], sems[(k + 1) % 2])

            pltpu.make_async_copy(z_h.at[pl.ds((lo + i) * N, N)],
                                  bufs[k % 2], sems[k % 2]).wait()

            def _acc(c, a, _k=k):
                return a + bufs[_k % 2][pl.ds(c * 16, 16)]

            acc = lax.fori_loop(0, N // 16, _acc,
                                jnp.zeros((16,), jnp.float32), unroll=8)
            res = jnp.where(lanes == k, jnp.sum(acc), res)

        wv[pl.ds(g * 16, 16)] = res

    pltpu.sync_copy(wv, w_h.at[pl.ds(lo, RPT)])


@functools.cache
def _rowsum_sc():
    return pl.kernel(
        _body_rs_sc,
        out_type=jax.ShapeDtypeStruct((N,), jnp.float32),
        mesh=_mesh(),
        compiler_params=pltpu.CompilerParams(needs_layout_passes=False),
        scratch_types=[
            pltpu.VMEM((N,), jnp.float32),
            pltpu.VMEM((N,), jnp.float32),
            pltpu.VMEM((RPT,), jnp.float32),
            pltpu.SemaphoreType.DMA,
            pltpu.SemaphoreType.DMA,
        ],
    )


FR = 256


def _body_fin(zf, zt, wc, wr, o):
    wcs = jnp.where(wc[...] > 0.0, wc[...], 1.0)
    wrs = jnp.where(wr[...] > 0.0, wr[...], 1.0)
    o[...] = 0.5 * (zf[...] / wcs + jnp.swapaxes(zt[...], 0, 1) / wrs)


_final = pl.pallas_call(
    _body_fin,
    out_shape=jax.ShapeDtypeStruct((N, N), jnp.float32),
    grid=(pl.cdiv(N, FR),),
    compiler_params=pltpu.CompilerParams(vmem_limit_bytes=127 * 1024 * 1024),
    in_specs=[
        pl.BlockSpec((FR, N), lambda i: (i, 0)),
        pl.BlockSpec((N, FR), lambda i: (0, i)),
        pl.BlockSpec((FR, 1), lambda i: (i, 0)),
        pl.BlockSpec((1, N), lambda i: (0, 0)),
    ],
    out_specs=pl.BlockSpec((FR, N), lambda i: (i, 0)),
)


def kernel(features, pos, edges, faces):
    del features, faces
    src = edges[0].astype(jnp.int32)
    dst = edges[1].astype(jnp.int32)
    px = jnp.asarray(pos[:, 0], jnp.float32)
    py = jnp.asarray(pos[:, 1], jnp.float32)
    pz = jnp.asarray(pos[:, 2], jnp.float32)

    z_ref = jax.new_ref(jnp.zeros((N * N,), jnp.float32))
    _kernel_a()(src, dst, px, py, pz, z_ref)
    z1 = jax.freeze(z_ref)
    w = _rowsum_sc()(z1)
    z2 = z1.reshape(N, N)
    return _final(z2, z2, w.reshape(N, 1), w.reshape(1, N))
